# baseline (speedup 1.0000x reference)
import jax
import jax.numpy as jnp
from jax import lax
from jax.experimental import pallas as pl
from jax.experimental.pallas import tpu as pltpu
from jax.experimental.pallas import tpu_sc as plsc

_NC, _NS, _L = 2, 16, 16
_K = 5


def _sc_body(col_hbm, row_hbm, out_hbm, colbuf, rowbuf, chunk, sem):
    b, hw, d2 = out_hbm.shape
    d = d2 // 2
    w = col_hbm.shape[0]
    wid = lax.axis_index("s") * _NC + lax.axis_index("c")
    pltpu.sync_copy(col_hbm, colbuf)
    pltpu.sync_copy(row_hbm.at[wid], rowbuf)
    rv = [rowbuf[pl.ds(k * _L, _L)] for k in range(d // _L)]
    for c in range(w):
        for k in range(d // _L):
            chunk[c, pl.ds(k * _L, _L)] = colbuf[c, pl.ds(k * _L, _L)]
            chunk[c, pl.ds(d + k * _L, _L)] = rv[k]
    copies = [
        pltpu.async_copy(chunk, out_hbm.at[i, pl.ds(wid * w, w), :], sem)
        for i in range(b)
    ]
    for cp in copies:
        cp.wait()


def _tc_body(col_ref, row_ref, out_ref, scratch, sem):
    w, d = col_ref.shape
    h = row_ref.shape[0]
    b = out_ref.shape[0]
    col = col_ref[...]
    row = row_ref[...]
    left = jnp.broadcast_to(col[None, :, :], (h, w, d)).reshape(h * w, d)
    right = jnp.broadcast_to(row[:, None, :], (h, w, d)).reshape(h * w, d)
    scratch[:, 0:d] = left
    scratch[:, d:2 * d] = right
    copies = [
        pltpu.make_async_copy(scratch, out_ref.at[i], sem.at[i])
        for i in range(b)
    ]
    for c in copies:
        c.start()
    for c in copies:
        c.wait()


def kernel(x, row_embed, col_embed):
    b = x.shape[0]
    h, w = x.shape[-3], x.shape[-2]
    d = row_embed.shape[1]
    assert h == _NC * _NS and w == h and d % _L == 0
    col = col_embed[:w]
    row = row_embed[:h]

    mesh = plsc.VectorSubcoreMesh(core_axis_name="c", subcore_axis_name="s")
    sc = pl.kernel(
        _sc_body,
        out_type=jax.ShapeDtypeStruct((_K, h * w, 2 * d), jnp.float32),
        mesh=mesh,
        scratch_types=[
            pltpu.VMEM((w, d), jnp.float32),
            pltpu.VMEM((d,), jnp.float32),
            pltpu.VMEM((w, 2 * d), jnp.float32),
            pltpu.SemaphoreType.DMA,
        ],
    )
    sc_part = sc(col, row)

    tc_part = pl.pallas_call(
        _tc_body,
        in_specs=[
            pl.BlockSpec((w, d), lambda: (0, 0)),
            pl.BlockSpec((h, d), lambda: (0, 0)),
        ],
        out_specs=pl.BlockSpec(memory_space=pl.ANY),
        out_shape=jax.ShapeDtypeStruct((b - _K, h * w, 2 * d), jnp.float32),
        scratch_shapes=[
            pltpu.VMEM((h * w, 2 * d), jnp.float32),
            pltpu.SemaphoreType.DMA((b - _K,)),
        ],
    )(col, row)

    return jnp.concatenate([sc_part, tc_part], axis=0)


# speedup vs baseline: 3.7492x; 3.7492x over previous
import jax
import jax.numpy as jnp
from jax.experimental import pallas as pl
from jax.experimental.pallas import tpu as pltpu


def _pos_body(col_ref, row_ref, out_ref, scratch, sem):
    w, d = col_ref.shape
    h = row_ref.shape[0]
    b = out_ref.shape[0]
    col = col_ref[...]
    row = row_ref[...]
    left = jnp.broadcast_to(col[None, :, :], (h, w, d)).reshape(h * w, d)
    right = jnp.broadcast_to(row[:, None, :], (h, w, d)).reshape(h * w, d)
    for j in range(2):
        scratch[j, :, 0:d] = left
        scratch[j, :, d:2 * d] = right
    copies = [
        pltpu.make_async_copy(scratch, out_ref.at[pl.ds(2 * i, 2)], sem.at[i])
        for i in range(b // 2)
    ]
    for c in copies:
        c.start()
    for c in copies:
        c.wait()


def kernel(x, row_embed, col_embed):
    b = x.shape[0]
    h, w = x.shape[-3], x.shape[-2]
    d = row_embed.shape[1]
    col = col_embed[:w]
    row = row_embed[:h]
    return pl.pallas_call(
        _pos_body,
        in_specs=[
            pl.BlockSpec((w, d), lambda: (0, 0)),
            pl.BlockSpec((h, d), lambda: (0, 0)),
        ],
        out_specs=pl.BlockSpec(memory_space=pl.ANY),
        out_shape=jax.ShapeDtypeStruct((b, h * w, 2 * d), jnp.float32),
        scratch_shapes=[
            pltpu.VMEM((2, h * w, 2 * d), jnp.float32),
            pltpu.SemaphoreType.DMA((b // 2,)),
        ],
    )(col, row)


# speedup vs baseline: 3.8073x; 1.0155x over previous
import jax
import jax.numpy as jnp
from jax.experimental import pallas as pl
from jax.experimental.pallas import tpu as pltpu

_SEG = 8


def _pos_body(col_ref, row_ref, out_ref, scratch, sem):
    w, d = col_ref.shape
    h = row_ref.shape[0]
    b = out_ref.shape[0]
    rs = h // _SEG
    col = col_ref[...]
    copies = []
    for s in range(_SEG):
        row_s = row_ref[pl.ds(s * rs, rs), :]
        left = jnp.broadcast_to(col[None, :, :], (rs, w, d)).reshape(rs * w, d)
        right = jnp.broadcast_to(row_s[:, None, :], (rs, w, d)).reshape(rs * w, d)
        lo = s * rs * w
        scratch[pl.ds(lo, rs * w), 0:d] = left
        scratch[pl.ds(lo, rs * w), d:2 * d] = right
        for i in range(b):
            cp = pltpu.make_async_copy(
                scratch.at[pl.ds(lo, rs * w)],
                out_ref.at[i, pl.ds(lo, rs * w), :],
                sem.at[i],
            )
            cp.start()
            copies.append(cp)
    for cp in copies:
        cp.wait()


def kernel(x, row_embed, col_embed):
    b = x.shape[0]
    h, w = x.shape[-3], x.shape[-2]
    d = row_embed.shape[1]
    col = col_embed[:w]
    row = row_embed[:h]
    return pl.pallas_call(
        _pos_body,
        in_specs=[
            pl.BlockSpec((w, d), lambda: (0, 0)),
            pl.BlockSpec((h, d), lambda: (0, 0)),
        ],
        out_specs=pl.BlockSpec(memory_space=pl.ANY),
        out_shape=jax.ShapeDtypeStruct((b, h * w, 2 * d), jnp.float32),
        scratch_shapes=[
            pltpu.VMEM((h * w, 2 * d), jnp.float32),
            pltpu.SemaphoreType.DMA((b,)),
        ],
    )(col, row)


# speedup vs baseline: 3.8785x; 1.0187x over previous
import jax
import jax.numpy as jnp
from jax.experimental import pallas as pl
from jax.experimental.pallas import tpu as pltpu

_SEG = 8


def _pos_body(col_ref, row_ref, out_ref, scratch, sem):
    w, d = col_ref.shape
    h = row_ref.shape[0]
    b = out_ref.shape[0]
    rs = h // _SEG
    col = col_ref[...]
    copies = []
    for s in range(_SEG):
        row_s = row_ref[pl.ds(s * rs, rs), :]
        left = jnp.broadcast_to(col[None, :, :], (rs, w, d)).reshape(rs * w, d)
        right = jnp.broadcast_to(row_s[:, None, :], (rs, w, d)).reshape(rs * w, d)
        lo = s * rs * w
        scratch[pl.ds(lo, rs * w), 0:d] = left
        scratch[pl.ds(lo, rs * w), d:2 * d] = right
        for i in range(b):
            cp = pltpu.make_async_copy(
                scratch.at[pl.ds(lo, rs * w)],
                out_ref.at[i, pl.ds(lo, rs * w), :],
                sem.at[i % 4],
            )
            cp.start()
            copies.append(cp)
    for cp in copies:
        cp.wait()


def kernel(x, row_embed, col_embed):
    b = x.shape[0]
    h, w = x.shape[-3], x.shape[-2]
    d = row_embed.shape[1]
    col = col_embed[:w]
    row = row_embed[:h]
    return pl.pallas_call(
        _pos_body,
        in_specs=[
            pl.BlockSpec((w, d), lambda: (0, 0)),
            pl.BlockSpec((h, d), lambda: (0, 0)),
        ],
        out_specs=pl.BlockSpec(memory_space=pl.ANY),
        out_shape=jax.ShapeDtypeStruct((b, h * w, 2 * d), jnp.float32),
        scratch_shapes=[
            pltpu.VMEM((h * w, 2 * d), jnp.float32),
            pltpu.SemaphoreType.DMA((4,)),
        ],
    )(col, row)
